# baseline (speedup 1.0000x reference)
import functools

import jax
import jax.numpy as jnp
from jax import lax
from jax.experimental import pallas as pl
from jax.experimental.pallas import tpu as pltpu
from jax.experimental.pallas import tpu_sc as plsc

N_DIM = 16
L = 16
SUBL = 8
LANE = 128


def _rsqrt(x):
    xi = plsc.bitcast(x, jnp.int32)
    y = plsc.bitcast(jnp.int32(0x5F3759DF) - (xi >> 1), jnp.float32)
    for _ in range(3):
        y = y * (1.5 - 0.5 * x * y * y)
    return y


def _tiled_flat(table):
    v = table.shape[0]
    vp = (v + LANE - 1) // LANE * LANE
    n_tiles = vp // LANE
    padded = jnp.pad(table, ((0, vp - v), (0, 0)))
    x = padded.T.reshape(N_DIM // SUBL, SUBL, n_tiles, LANE)
    return x.transpose(0, 2, 1, 3).reshape(-1), n_tiles


def _bias_flat(bias):
    v = bias.shape[0]
    vp = (v + 1023) // 1024 * 1024
    return jnp.pad(bias, ((0, vp - v), (0, 0))).reshape(-1)


def _make_stage1(batch):
    info = plsc.get_sparse_core_info()
    nc, ns = info.num_cores, info.num_subcores
    nw = nc * ns
    bpw = batch // nw
    mesh = plsc.VectorSubcoreMesh(core_axis_name="c", subcore_axis_name="s")

    @functools.partial(
        pl.kernel,
        mesh=mesh,
        compiler_params=pltpu.CompilerParams(
            needs_layout_passes=False, use_tc_tiling_on_sc=False),
        out_type=(jax.ShapeDtypeStruct((batch,), jnp.int32),
                  jax.ShapeDtypeStruct((batch,), jnp.int32),
                  jax.ShapeDtypeStruct((batch,), jnp.float32)),
        scratch_types=[
            pltpu.VMEM((bpw,), jnp.int32),
            pltpu.VMEM((bpw,), jnp.int32),
            pltpu.VMEM((bpw,), jnp.int32),
            pltpu.VMEM((bpw,), jnp.int32),
            pltpu.VMEM((bpw,), jnp.float32),
            pltpu.VMEM((bpw,), jnp.float32),
            pltpu.SemaphoreType.DMA,
        ],
    )
    def k1(u_hbm, i_hbm, ub_hbm, ib_hbm, tu_hbm, ti_hbm, bsum_hbm,
           u_v, i_v, tu_v, ti_v, ub_v, ib_v, sem):
        wid = lax.axis_index("s") * nc + lax.axis_index("c")
        base = wid * bpw
        cp_u = pltpu.async_copy(u_hbm.at[pl.ds(base, bpw)], u_v, sem)
        cp_i = pltpu.async_copy(i_hbm.at[pl.ds(base, bpw)], i_v, sem)
        cp_u.wait()
        cp_i.wait()
        for j in range(bpw // L):
            sl = pl.ds(j * L, L)
            uvec = u_v[sl]
            ivec = i_v[sl]
            tu_v[sl] = (uvec >> 7) * (SUBL * LANE) + (uvec & (LANE - 1))
            ti_v[sl] = (ivec >> 7) * (SUBL * LANE) + (ivec & (LANE - 1))
        cps = [
            pltpu.async_copy(ub_hbm.at[u_v], ub_v, sem),
            pltpu.async_copy(ib_hbm.at[i_v], ib_v, sem),
        ]
        for cp in cps:
            cp.wait()
        for j in range(bpw // L):
            sl = pl.ds(j * L, L)
            ub_v[sl] = ub_v[sl] + ib_v[sl]
        cps = [
            pltpu.async_copy(tu_v, tu_hbm.at[pl.ds(base, bpw)], sem),
            pltpu.async_copy(ti_v, ti_hbm.at[pl.ds(base, bpw)], sem),
            pltpu.async_copy(ub_v, bsum_hbm.at[pl.ds(base, bpw)], sem),
        ]
        for cp in cps:
            cp.wait()

    return k1


def _make_stage2(batch, n_tiles):
    info = plsc.get_sparse_core_info()
    nc, ns = info.num_cores, info.num_subcores
    nw = nc * ns
    assert batch % (nw * L) == 0
    bpw = batch // nw
    n_groups = bpw // L
    band = n_tiles * SUBL * LANE
    flat_len = (N_DIM // SUBL) * band
    mesh = plsc.VectorSubcoreMesh(core_axis_name="c", subcore_axis_name="s")

    @functools.partial(
        pl.kernel,
        mesh=mesh,
        compiler_params=pltpu.CompilerParams(
            needs_layout_passes=False, use_tc_tiling_on_sc=False),
        out_type=jax.ShapeDtypeStruct((batch,), jnp.float32),
        scratch_types=[
            pltpu.VMEM((bpw,), jnp.int32),
            pltpu.VMEM((bpw,), jnp.int32),
            pltpu.VMEM((N_DIM * bpw,), jnp.float32),
            pltpu.VMEM((N_DIM * bpw,), jnp.float32),
            pltpu.VMEM((bpw,), jnp.float32),
            pltpu.VMEM((L,), jnp.float32),
            pltpu.VMEM((bpw,), jnp.float32),
            pltpu.SemaphoreType.DMA,
        ],
    )
    def k2(uv_hbm, iv_hbm, tu_hbm, ti_hbm, bsum_hbm, gb_hbm, out_hbm,
           tu_v, ti_v, pu_v, pi_v, bs_v, gb_v, out_v, sem):
        wid = lax.axis_index("s") * nc + lax.axis_index("c")
        base = wid * bpw
        cps = [
            pltpu.async_copy(tu_hbm.at[pl.ds(base, bpw)], tu_v, sem),
            pltpu.async_copy(ti_hbm.at[pl.ds(base, bpw)], ti_v, sem),
            pltpu.async_copy(bsum_hbm.at[pl.ds(base, bpw)], bs_v, sem),
        ]
        pltpu.sync_copy(gb_hbm, gb_v.at[pl.ds(0, 1)])
        for cp in cps:
            cp.wait()

        cps = []
        for d in range(N_DIM):
            off = (d // SUBL) * band + (d % SUBL) * LANE
            span = flat_len - off
            cps.append(pltpu.async_copy(
                uv_hbm.at[pl.ds(off, span)].at[tu_v],
                pu_v.at[pl.ds(d * bpw, bpw)], sem))
            cps.append(pltpu.async_copy(
                iv_hbm.at[pl.ds(off, span)].at[ti_v],
                pi_v.at[pl.ds(d * bpw, bpw)], sem))
        for cp in cps:
            cp.wait()

        gb = gb_v[...][0]

        def group(g, carry):
            bs = bs_v[pl.ds(g * L, L)]
            s_u = jnp.zeros((L,), jnp.float32)
            s_i = jnp.zeros((L,), jnp.float32)
            t = jnp.zeros((L,), jnp.float32)
            for d in range(N_DIM):
                lu = pu_v[pl.ds(d * bpw + g * L, L)]
                li = pi_v[pl.ds(d * bpw + g * L, L)]
                eu = jnp.exp(0.5 * lu)
                ei = jnp.exp(0.5 * li)
                s_u = s_u + eu * eu
                s_i = s_i + ei * ei
                t = t + eu * ei
            bc = t * _rsqrt(s_u * s_i)
            z = jnp.maximum(1.0 - bc, 1e-36)
            intx = z * _rsqrt(z)
            out_v[pl.ds(g * L, L)] = bs + intx + gb
            return carry

        lax.fori_loop(0, n_groups, group, 0)
        pltpu.sync_copy(out_v, out_hbm.at[pl.ds(base, bpw)])

    return k2


def kernel(u, i, user_bias, user_vect, item_bias, item_vect, glob_bias):
    batch = u.shape[0]
    uv_flat, n_tiles = _tiled_flat(user_vect)
    iv_flat, _ = _tiled_flat(item_vect)
    k1 = _make_stage1(batch)
    k2 = _make_stage2(batch, n_tiles)
    tu, ti, bsum = k1(u.astype(jnp.int32), i.astype(jnp.int32),
                      _bias_flat(user_bias), _bias_flat(item_bias))
    return k2(uv_flat, iv_flat, tu, ti, bsum, glob_bias)
